# Initial kernel scaffold; baseline (speedup 1.0000x reference)
import jax
import jax.numpy as jnp
from jax.experimental import pallas as pl


def kernel(x, router_W, fc1_W, fc1_b, fc2_W, fc2_b, out_W, out_b):
    raise NotImplementedError("write your pallas kernel here")


# speedup vs baseline: 5.5444x; 5.5444x over previous
import functools

import jax
import jax.numpy as jnp
from jax.experimental import pallas as pl
from jax.experimental.pallas import tpu as pltpu


def _router_body(x_ref, rw_ref, topi_ref, aux_ref):
    b, n, d = x_ref.shape
    e = rw_ref.shape[0]
    rows = [jnp.mean(x_ref[i], axis=0, keepdims=True) for i in range(b)]
    xm = jnp.concatenate(rows, axis=0)
    logits = jax.lax.dot_general(
        xm, rw_ref[...], (((1,), (1,)), ((), ())),
        preferred_element_type=jnp.float32)
    lmax = jnp.max(logits, axis=-1, keepdims=True)
    ex = jnp.exp(logits - lmax)
    probs = ex / jnp.sum(ex, axis=-1, keepdims=True)
    ii = jax.lax.broadcasted_iota(jnp.int32, (b, e), 1)
    pmax = jnp.max(probs, axis=-1, keepdims=True)
    top1 = jnp.min(jnp.where(probs == pmax, ii, e), axis=-1, keepdims=True)
    topi_ref[...] = top1
    onehot = (ii == top1).astype(jnp.float32)
    pm = jnp.mean(probs, axis=0, keepdims=True)
    em = jnp.mean(onehot, axis=0, keepdims=True)
    aux_ref[...] = e * jnp.sum(pm * em, axis=(0, 1), keepdims=True)


def _mixer_body(topi_ref, x_ref, f1w_ref, f1b_ref, f2w_ref, f2b_ref,
                outw_ref, outb_ref, out_ref, *, num_dt):
    dt = pl.program_id(1)
    xs = x_ref[0]
    mu = jnp.mean(xs, axis=0, keepdims=True)
    var = jnp.mean((xs - mu) ** 2, axis=0, keepdims=True)
    xn = (xs - mu) / jnp.sqrt(var + 1e-5)
    h = jax.lax.dot_general(
        xn, f1w_ref[0], (((0,), (1,)), ((), ())),
        preferred_element_type=jnp.float32)
    h = jax.nn.gelu(h + f1b_ref[0], approximate=True)
    y = jax.lax.dot_general(
        h, f2w_ref[0], (((1,), (1,)), ((), ())),
        preferred_element_type=jnp.float32)
    y = y + f2b_ref[0]
    contrib = jax.lax.dot_general(
        y, outw_ref[...], (((0,), (1,)), ((), ())),
        preferred_element_type=jnp.float32)

    @pl.when(dt == 0)
    def _():
        out_ref[0] = contrib

    @pl.when(dt > 0)
    def _():
        out_ref[0] += contrib

    @pl.when(dt == num_dt - 1)
    def _():
        out_ref[0] += outb_ref[...]


@jax.jit
def kernel(x, router_W, fc1_W, fc1_b, fc2_W, fc2_b, out_W, out_b):
    B, N, D = x.shape
    E, H, _ = fc1_W.shape
    TD = 256
    num_dt = D // TD

    topi, aux = pl.pallas_call(
        _router_body,
        out_shape=(
            jax.ShapeDtypeStruct((B, 1), jnp.int32),
            jax.ShapeDtypeStruct((1, 1), jnp.float32),
        ),
    )(x, router_W)
    topi_flat = topi.reshape(B)

    f1b3 = fc1_b.reshape(E, 1, H)
    f2b3 = fc2_b.reshape(E, 1, N)
    outb2 = out_b.reshape(1, D)

    grid_spec = pltpu.PrefetchScalarGridSpec(
        num_scalar_prefetch=1,
        grid=(B, num_dt),
        in_specs=[
            pl.BlockSpec((1, N, TD), lambda b, d, ti: (b, 0, d)),
            pl.BlockSpec((1, H, N), lambda b, d, ti: (ti[b], 0, 0)),
            pl.BlockSpec((1, 1, H), lambda b, d, ti: (ti[b], 0, 0)),
            pl.BlockSpec((1, N, H), lambda b, d, ti: (ti[b], 0, 0)),
            pl.BlockSpec((1, 1, N), lambda b, d, ti: (ti[b], 0, 0)),
            pl.BlockSpec((D, TD), lambda b, d, ti: (0, d)),
            pl.BlockSpec((1, D), lambda b, d, ti: (0, 0)),
        ],
        out_specs=pl.BlockSpec((1, N, D), lambda b, d, ti: (b, 0, 0)),
    )
    out = pl.pallas_call(
        functools.partial(_mixer_body, num_dt=num_dt),
        grid_spec=grid_spec,
        out_shape=jax.ShapeDtypeStruct((B, N, D), jnp.float32),
    )(topi_flat, x, fc1_W, f1b3, fc2_W, f2b3, out_W, outb2)

    return out, aux[0, 0]
